# Initial kernel scaffold; baseline (speedup 1.0000x reference)
import jax
import jax.numpy as jnp
from jax.experimental import pallas as pl


def kernel(features, edge_index, params):
    raise NotImplementedError("write your pallas kernel here")


# speedup vs baseline: 14.5013x; 14.5013x over previous
import functools

import jax
import jax.numpy as jnp
from jax import lax
from jax.experimental import pallas as pl
from jax.experimental.pallas import tpu as pltpu
from jax.experimental.pallas import tpu_sc as plsc

N = 10000
E = 320000
DF = 128
H = 32
NL = 5
NC, NS = 2, 16
EW = E // NS
CH = 125
NCH = EW // CH
NBUF = 4
NP = 10240
RT = NP // NS

f32 = jnp.float32


def _edge_loop(table, gat_v, sct_v, rows, gsems, ssems, agg_s):
    def body(k, carry):
        base = k * NBUF
        gets = [pltpu.async_copy(table.at[gat_v.at[base + b]], rows[b],
                                 gsems[b]) for b in range(NBUF)]
        puts = []
        for b in range(NBUF):
            gets[b].wait()
            puts.append(pltpu.async_copy(rows[b], agg_s.at[sct_v.at[base + b]],
                                         ssems[b], add=True))
        for b in range(NBUF):
            puts[b].wait()
        return carry
    lax.fori_loop(0, NCH // NBUF, body, 0)


def _sc_agg_body(hf, hr, srcm, dstm, zeros_hbm, aggf_out, aggr_out,
                 src_v, dst_v, rows0, rows1, rows2, rows3,
                 gs0, gs1, gs2, gs3, ss0, ss1, ss2, ss3, agg_s):
    rows = [rows0, rows1, rows2, rows3]
    gsems = [gs0, gs1, gs2, gs3]
    ssems = [ss0, ss1, ss2, ss3]
    c = lax.axis_index("c")
    s = lax.axis_index("s")
    pltpu.sync_copy(srcm.at[s], src_v)
    pltpu.sync_copy(dstm.at[s], dst_v)
    pltpu.sync_copy(zeros_hbm.at[pl.ds(s * RT, RT)], agg_s.at[pl.ds(s * RT, RT)])
    plsc.subcore_barrier()

    @pl.when(c == 0)
    def _fwd():
        _edge_loop(hf, src_v, dst_v, rows, gsems, ssems, agg_s)

    @pl.when(c == 1)
    def _rev():
        _edge_loop(hr, dst_v, src_v, rows, gsems, ssems, agg_s)

    plsc.subcore_barrier()

    @pl.when(c == 0)
    def _outf():
        pltpu.sync_copy(agg_s.at[pl.ds(s * RT, RT)], aggf_out.at[pl.ds(s * RT, RT)])

    @pl.when(c == 1)
    def _outr():
        pltpu.sync_copy(agg_s.at[pl.ds(s * RT, RT)], aggr_out.at[pl.ds(s * RT, RT)])


def _ones_loop(idx_v, ones_v, sem, agg_s):
    def body(k, carry):
        base = k * NBUF
        puts = [pltpu.async_copy(ones_v, agg_s.at[idx_v.at[base + b]],
                                 sem, add=True) for b in range(NBUF)]
        for p in puts:
            p.wait()
        return carry
    lax.fori_loop(0, NCH // NBUF, body, 0)


def _sc_deg_body(srcm, dstm, ones_hbm, zeros_hbm, dego_out, degi_out,
                 src_v, dst_v, ones_v, sem, agg_s):
    c = lax.axis_index("c")
    s = lax.axis_index("s")
    pltpu.sync_copy(srcm.at[s], src_v)
    pltpu.sync_copy(dstm.at[s], dst_v)
    pltpu.sync_copy(ones_hbm, ones_v)
    pltpu.sync_copy(zeros_hbm.at[pl.ds(s * RT, RT)], agg_s.at[pl.ds(s * RT, RT)])
    plsc.subcore_barrier()

    @pl.when(c == 0)
    def _fwd():
        _ones_loop(src_v, ones_v, sem, agg_s)

    @pl.when(c == 1)
    def _rev():
        _ones_loop(dst_v, ones_v, sem, agg_s)

    plsc.subcore_barrier()

    @pl.when(c == 0)
    def _outo():
        pltpu.sync_copy(agg_s.at[pl.ds(s * RT, RT)], dego_out.at[pl.ds(s * RT, RT)])

    @pl.when(c == 1)
    def _outi():
        pltpu.sync_copy(agg_s.at[pl.ds(s * RT, RT)], degi_out.at[pl.ds(s * RT, RT)])


@functools.cache
def _get_sc_kernels():
    mesh = plsc.VectorSubcoreMesh(core_axis_name="c", subcore_axis_name="s",
                                  num_cores=NC, num_subcores=NS)
    deg = pl.kernel(
        _sc_deg_body,
        [jax.ShapeDtypeStruct((NP, H), f32),
         jax.ShapeDtypeStruct((NP, H), f32)],
        mesh=mesh,
        compiler_params=pltpu.CompilerParams(use_tc_tiling_on_sc=False),
        scratch_types=[
            pltpu.VMEM((NCH, CH), jnp.int32),
            pltpu.VMEM((NCH, CH), jnp.int32),
            pltpu.VMEM((CH, H), f32),
            pltpu.SemaphoreType.DMA,
            pltpu.VMEM_SHARED((NP, H), f32),
        ],
    )
    agg = pl.kernel(
        _sc_agg_body,
        [jax.ShapeDtypeStruct((NP, H), f32),
         jax.ShapeDtypeStruct((NP, H), f32)],
        mesh=mesh,
        compiler_params=pltpu.CompilerParams(use_tc_tiling_on_sc=False),
        scratch_types=(
            [pltpu.VMEM((NCH, CH), jnp.int32),
             pltpu.VMEM((NCH, CH), jnp.int32)]
            + [pltpu.VMEM((CH, H), f32)] * NBUF
            + [pltpu.SemaphoreType.DMA] * (2 * NBUF)
            + [pltpu.VMEM_SHARED((NP, H), f32)]
        ),
    )
    return deg, agg


_RB = 1000
_NB = N // _RB


def _dot1(x, w):
    bf16 = jnp.bfloat16
    return jnp.dot(x.astype(bf16), w.astype(bf16), preferred_element_type=f32)


def _scl(deg):
    return lax.rsqrt(jnp.maximum(deg, 1.0))


def _tc_pre_body(feat, wc0, dego, degi, hf_o, hr_o):
    x = feat[...]
    w = wc0[...]
    ds1 = _scl(dego[...])[:, 0:1]
    di1 = _scl(degi[...])[:, 0:1]
    hf_o[...] = _dot1(x * ds1, w)
    hr_o[...] = _dot1(x * di1, w)


_tc_pre = pl.pallas_call(
    _tc_pre_body,
    grid=(N // _RB,),
    in_specs=[
        pl.BlockSpec((_RB, DF), lambda i: (i, 0)),
        pl.BlockSpec((DF, H), lambda i: (0, 0)),
        pl.BlockSpec((_RB, H), lambda i: (i, 0)),
        pl.BlockSpec((_RB, H), lambda i: (i, 0)),
    ],
    out_specs=[
        pl.BlockSpec((_RB, H), lambda i: (i, 0)),
        pl.BlockSpec((_RB, H), lambda i: (i, 0)),
    ],
    out_shape=[jax.ShapeDtypeStruct((N, H), f32),
               jax.ShapeDtypeStruct((N, H), f32)],
)


_blk = pl.BlockSpec((_RB, H), lambda p, i: (i, 0))
_w64 = pl.BlockSpec((2 * H, H), lambda p, i: (0, 0))
_w32 = pl.BlockSpec((H, H), lambda p, i: (0, 0))
_v32 = pl.BlockSpec((1, H), lambda p, i: (0, 0))
_oblk = pl.BlockSpec((_RB, H), lambda p, i: (jnp.where(p == 2, i, 0), 0))


def _tc_mid_body(has_skip_in, store_skip, *refs):
    (aggf, aggr, dego, degi, wl, g, b) = refs[:7]
    k = 7
    if has_skip_in:
        skip_in = refs[k]; k += 1
    wc_next = refs[k]; k += 1
    outs = []
    while k < len(refs) - 3:
        outs.append(refs[k]); k += 1
    y_s, sum_s, sq_s = refs[-3:]

    p = pl.program_id(0)
    i = pl.program_id(1)
    dsv = _scl(dego[...])
    div = _scl(degi[...])

    @pl.when(p == 0)
    def _phase0():
        @pl.when(i == 0)
        def _init():
            sum_s[...] = jnp.zeros((1, H), f32)
            sq_s[...] = jnp.zeros((1, H), f32)
        cat = jnp.concatenate([aggf[...] * div, aggr[...] * dsv], axis=1)
        yblk = _dot1(cat, wl[...])
        y_s[pl.ds(i * _RB, _RB), :] = yblk
        sum_s[...] += jnp.sum(yblk, axis=0, keepdims=True)

    @pl.when(p == 1)
    def _phase1():
        mu = sum_s[...] * (1.0 / N)
        d = y_s[pl.ds(i * _RB, _RB), :] - mu
        sq_s[...] += jnp.sum(d * d, axis=0, keepdims=True)

    @pl.when(p == 2)
    def _phase2():
        mu = sum_s[...] * (1.0 / N)
        var = sq_s[...] * (1.0 / N)
        y = y_s[pl.ds(i * _RB, _RB), :]
        y = (y - mu) * lax.rsqrt(var + 1e-5) * g[...] + b[...]
        if has_skip_in:
            y = y + skip_in[...]
        y = jnp.maximum(y, 0.0)
        j = 0
        if store_skip:
            outs[j][...] = y; j += 1
        w = wc_next[...]
        outs[j][...] = _dot1(y * dsv, w)
        outs[j + 1][...] = _dot1(y * div, w)


def _make_mid(has_skip_in, store_skip):
    in_specs = [_blk, _blk, _blk, _blk, _w64, _v32, _v32]
    if has_skip_in:
        in_specs.append(_blk)
    in_specs.append(_w32)
    n_out = (1 if store_skip else 0) + 2
    out_specs = [_oblk] * n_out
    return pl.pallas_call(
        functools.partial(_tc_mid_body, has_skip_in, store_skip),
        grid=(3, _NB),
        in_specs=in_specs,
        out_specs=out_specs,
        out_shape=[jax.ShapeDtypeStruct((N, H), f32)] * n_out,
        scratch_shapes=[pltpu.VMEM((N, H), f32),
                        pltpu.VMEM((1, H), f32),
                        pltpu.VMEM((1, H), f32)],
    )


_tc_mids = [_make_mid(False, True),
            _make_mid(False, True),
            _make_mid(True, True),
            _make_mid(True, False)]


def _tc_fin_body(aggf, aggr, dego, degi, wl, g, b, skip_in, wout, bout,
                 out, y_s, sum_s, sq_s):
    p = pl.program_id(0)
    i = pl.program_id(1)

    @pl.when(p == 0)
    def _phase0():
        @pl.when(i == 0)
        def _init():
            sum_s[...] = jnp.zeros((1, H), f32)
            sq_s[...] = jnp.zeros((1, H), f32)
        cat = jnp.concatenate([aggf[...] * _scl(degi[...]),
                               aggr[...] * _scl(dego[...])], axis=1)
        yblk = _dot1(cat, wl[...]) + skip_in[...]
        y_s[pl.ds(i * _RB, _RB), :] = yblk
        sum_s[...] += jnp.sum(yblk, axis=0, keepdims=True)

    @pl.when(p == 1)
    def _phase1():
        mu = sum_s[...] * (1.0 / N)
        d = y_s[pl.ds(i * _RB, _RB), :] - mu
        sq_s[...] += jnp.sum(d * d, axis=0, keepdims=True)

    @pl.when(p == 2)
    def _phase2():
        mu = sum_s[...] * (1.0 / N)
        var = sq_s[...] * (1.0 / N)
        y = y_s[pl.ds(i * _RB, _RB), :]
        y = (y - mu) * lax.rsqrt(var + 1e-5) * g[...] + b[...]
        out[...] = _dot1(y, wout[...]) + bout[...]


_tc_fin = pl.pallas_call(
    _tc_fin_body,
    grid=(3, _NB),
    in_specs=[_blk, _blk, _blk, _blk, _w64, _v32, _v32, _blk,
              pl.BlockSpec((H, 1), lambda p, i: (0, 0)),
              pl.BlockSpec((1, 1), lambda p, i: (0, 0))],
    out_specs=pl.BlockSpec((_RB, 1), lambda p, i: (jnp.where(p == 2, i, 0), 0)),
    out_shape=jax.ShapeDtypeStruct((N, 1), f32),
    scratch_shapes=[pltpu.VMEM((N, H), f32),
                    pltpu.VMEM((1, H), f32),
                    pltpu.VMEM((1, H), f32)],
)


def kernel(features, edge_index, params):
    src = edge_index[0].astype(jnp.int32)
    dst = edge_index[1].astype(jnp.int32)
    srcm = src.reshape(NS, NCH, CH)
    dstm = dst.reshape(NS, NCH, CH)
    zeros = jnp.zeros((NP, H), f32)
    ones = jnp.ones((CH, H), f32)

    sc_deg, sc_agg = _get_sc_kernels()
    dego, degi = sc_deg(srcm, dstm, ones, zeros)
    hf, hr = _tc_pre(features, params['Wc0'], dego, degi)

    skips = {}
    for i in range(NL):
        aggf, aggr = sc_agg(hf, hr, srcm, dstm, zeros)
        g = params['g%d' % i].reshape(1, H)
        b = params['b%d' % i].reshape(1, H)
        wl = params['Wl%d' % i]
        if i < NL - 1:
            args = [aggf, aggr, dego, degi, wl, g, b]
            if i >= 2:
                args.append(skips[i - 2])
            args.append(params['Wc%d' % (i + 1)])
            res = _tc_mids[i](*args)
            if i <= 2:
                skips[i] = res[0]
                hf, hr = res[1], res[2]
            else:
                hf, hr = res[0], res[1]
        else:
            out = _tc_fin(aggf, aggr, dego, degi, wl, g, b,
                          skips[i - 2], params['Wout'],
                          params['bout'].reshape(1, 1))
    return out
